# baseline (speedup 1.0000x reference)
import functools

import jax
import jax.numpy as jnp
from jax import lax
from jax.experimental import pallas as pl
from jax.experimental.pallas import tpu as pltpu
from jax.experimental.pallas import tpu_sc as plsc

BATCH = 4096
SEQ = 200
D = 64
NC, NS, LANES = 2, 16, 16
NW = NC * NS
BBLK = BATCH // NW
DV = D // LANES
SB = 8
SEQT = SEQ // SB


def _sc_body(x_hbm, w_hbm, p_hbm, out_hbm,
             x_v, p_v, g0, g1, g2, g3, o0, o1,
             gs0, gs1, gs2, gs3, ss0, ss1):
    gbufs = (g0, g1, g2, g3)
    obufs = (o0, o1)
    gsems = (gs0, gs1, gs2, gs3)
    ssems = (ss0, ss1)
    wid = lax.axis_index("s") * NC + lax.axis_index("c")
    pltpu.sync_copy(p_hbm, p_v)
    pltpu.sync_copy(x_hbm.at[:, wid], x_v)

    def issue_gather(s, k):
        pltpu.async_copy(
            w_hbm.at[x_v.at[s // SB, lax.rem(s, SB)]], gbufs[k], gsems[k]
        )

    def drain_gather(k):
        pltpu.make_async_copy(
            w_hbm.at[x_v.at[0, 0]], gbufs[k], gsems[k]
        ).wait()

    def issue_store(s, k):
        pltpu.async_copy(obufs[k], out_hbm.at[s, :, wid], ssems[k])

    def wait_store(k):
        pltpu.make_async_copy(obufs[k], out_hbm.at[0, :, wid], ssems[k]).wait()

    lane = lax.iota(jnp.int32, LANES)
    dvecs = [lane + (c * LANES) for c in range(DV)]
    dhi = [dvecs[c] // SB for c in range(DV)]
    dlo = [lax.rem(dvecs[c], SB) for c in range(DV)]
    rots = [lax.rem(lane + k, LANES) for k in range(LANES)]

    issue_gather(0, 0)
    issue_gather(1, 1)
    issue_gather(2, 2)

    @pl.loop(0, SEQ, step=4)
    def _pos(s0):
        for kb in range(4):
            s = s0 + kb
            gk = kb
            ok = kb % 2

            @pl.when(s + 3 < SEQ)
            def _():
                issue_gather(s + 3, (kb + 3) % 4)

            @pl.when(s >= 2)
            def _():
                wait_store(ok)

            drain_gather(gk)

            pvec = [p_v[s, pl.ds(c * LANES, LANES)] for c in range(DV)]

            @pl.loop(0, BBLK // LANES, unroll=2)
            def _blk(jb):
                jv = jnp.full((LANES,), jb * LANES, dtype=jnp.int32)
                rvs = [jv + rots[k] for k in range(LANES)]
                for k in range(LANES):
                    vals = [
                        plsc.load_gather(gbufs[gk], [rvs[k], dvecs[c]])
                        for c in range(DV)
                    ]
                    vals = [vals[c] + pvec[c] for c in range(DV)]
                    for c in range(DV):
                        plsc.store_scatter(
                            obufs[ok], [dhi[c], dlo[c], rvs[k]], vals[c]
                        )

            issue_store(s, ok)

    wait_store(0)
    wait_store(1)


_sc_kernel = functools.partial(
    pl.kernel,
    out_type=jax.ShapeDtypeStruct((SEQ, SB, NW, SB, BBLK), jnp.float32),
    mesh=plsc.VectorSubcoreMesh(core_axis_name="c", subcore_axis_name="s"),
    scratch_types=[
        pltpu.VMEM((SEQT, SB, BBLK), jnp.int32),
        pltpu.VMEM((SEQ, D), jnp.float32),
        pltpu.VMEM((BBLK, D), jnp.float32),
        pltpu.VMEM((BBLK, D), jnp.float32),
        pltpu.VMEM((BBLK, D), jnp.float32),
        pltpu.VMEM((BBLK, D), jnp.float32),
        pltpu.VMEM((SB, SB, BBLK), jnp.float32),
        pltpu.VMEM((SB, SB, BBLK), jnp.float32),
        pltpu.SemaphoreType.DMA,
        pltpu.SemaphoreType.DMA,
        pltpu.SemaphoreType.DMA,
        pltpu.SemaphoreType.DMA,
        pltpu.SemaphoreType.DMA,
        pltpu.SemaphoreType.DMA,
    ],
    compiler_params=pltpu.CompilerParams(
        use_tc_tiling_on_sc=False, needs_layout_passes=False
    ),
)(_sc_body)


@jax.jit
def kernel(x, W, P):
    x4 = x.T.reshape(SEQT, SB, NW, BBLK).transpose(0, 2, 1, 3)
    out5 = _sc_kernel(x4, W, P)
    return out5.transpose(2, 4, 0, 1, 3).reshape(BATCH, SEQ, D)


# speedup vs baseline: 1.4604x; 1.4604x over previous
import functools

import jax
import jax.numpy as jnp
from jax import lax
from jax.experimental import pallas as pl
from jax.experimental.pallas import tpu as pltpu
from jax.experimental.pallas import tpu_sc as plsc

BATCH = 4096
SEQ = 200
D = 64
NC, NS, LANES = 2, 16, 16
NW = NC * NS
BBLK = BATCH // NW
DV = D // LANES
SB = 8
SEQT = SEQ // SB


def _sc_body(x_hbm, w_hbm, p_hbm, out_hbm,
             x_v, p_v, g0, g1, o0, o1, gs0, gs1, ss0, ss1):
    gbufs = (g0, g1)
    obufs = (o0, o1)
    gsems = (gs0, gs1)
    ssems = (ss0, ss1)
    wid = lax.axis_index("s") * NC + lax.axis_index("c")
    pltpu.sync_copy(p_hbm, p_v)
    pltpu.sync_copy(x_hbm.at[:, wid], x_v)

    def issue_gather(s, k):
        pltpu.async_copy(
            w_hbm.at[x_v.at[s // SB, lax.rem(s, SB)]], gbufs[k], gsems[k]
        )

    def drain_gather(k):
        pltpu.make_async_copy(
            w_hbm.at[x_v.at[0, 0]], gbufs[k], gsems[k]
        ).wait()

    def issue_store(s, k):
        pltpu.async_copy(obufs[k], out_hbm.at[s, :, wid], ssems[k])

    def wait_store(k):
        pltpu.make_async_copy(obufs[k], out_hbm.at[0, :, wid], ssems[k]).wait()

    lane = lax.iota(jnp.int32, LANES)
    dvecs = [lane + (c * LANES) for c in range(DV)]
    dhi = [dvecs[c] // SB for c in range(DV)]
    dlo = [lax.rem(dvecs[c], SB) for c in range(DV)]
    rots = [lax.rem(lane + k, LANES) for k in range(LANES)]

    issue_gather(0, 0)

    @pl.loop(0, SEQ, step=2)
    def _pos(s0):
        for kb in range(2):
            s = s0 + kb
            gk = kb
            ok = kb

            @pl.when(s + 1 < SEQ)
            def _():
                issue_gather(s + 1, 1 - kb)
                @pl.when(s >= 1)
                def _():
                    wait_store(1 - kb)

            drain_gather(gk)

            pvec = [p_v[s, pl.ds(c * LANES, LANES)] for c in range(DV)]

            @pl.loop(0, BBLK // LANES)
            def _blk(jb):
                jv = jnp.full((LANES,), jb * LANES, dtype=jnp.int32)
                rvs = [jv + rots[k] for k in range(LANES)]
                for k in range(LANES):
                    vals = [
                        plsc.load_gather(gbufs[gk], [rvs[k], dvecs[c]])
                        for c in range(DV)
                    ]
                    vals = [vals[c] + pvec[c] for c in range(DV)]
                    for c in range(DV):
                        plsc.store_scatter(
                            obufs[ok], [dhi[c], dlo[c], rvs[k]], vals[c]
                        )

            issue_store(s, ok)

    wait_store(0)
    wait_store(1)


_sc_kernel = functools.partial(
    pl.kernel,
    out_type=jax.ShapeDtypeStruct((SEQ, SB, NW, SB, BBLK), jnp.float32),
    mesh=plsc.VectorSubcoreMesh(core_axis_name="c", subcore_axis_name="s"),
    scratch_types=[
        pltpu.VMEM((SEQT, SB, BBLK), jnp.int32),
        pltpu.VMEM((SEQ, D), jnp.float32),
        pltpu.VMEM((BBLK, D), jnp.float32),
        pltpu.VMEM((BBLK, D), jnp.float32),
        pltpu.VMEM((SB, SB, BBLK), jnp.float32),
        pltpu.VMEM((SB, SB, BBLK), jnp.float32),
        pltpu.SemaphoreType.DMA,
        pltpu.SemaphoreType.DMA,
        pltpu.SemaphoreType.DMA,
        pltpu.SemaphoreType.DMA,
    ],
    compiler_params=pltpu.CompilerParams(
        use_tc_tiling_on_sc=False, needs_layout_passes=False
    ),
)(_sc_body)


@jax.jit
def kernel(x, W, P):
    x4 = x.T.reshape(SEQT, SB, NW, BBLK).transpose(0, 2, 1, 3)
    out5 = _sc_kernel(x4, W, P)
    return out5.transpose(2, 4, 0, 1, 3).reshape(BATCH, SEQ, D)
